# baseline (speedup 1.0000x reference)
import functools

import jax
import jax.numpy as jnp
from jax import lax
from jax.experimental import pallas as pl
from jax.experimental.pallas import tpu as pltpu
from jax.experimental.pallas import tpu_sc as plsc

VOCAB = 100000
HIDDEN = 64
MAXPOS = 512
TRI = 20
B = 4096
SEQ = 20

NC = 2
NS = 16
NW = NC * NS
ROWS_PER_W = B // NW
CB = 16
CS = CB * SEQ
NCHUNK = ROWS_PER_W // CB


def _body(ids_hbm, pos_hbm, tri_hbm, post_hbm, out_hbm,
          ids2_v, idxT2_v, acc2_v, pidx2_v, addsem, psem, outsem):
    wid = lax.axis_index("s") * NC + lax.axis_index("c")
    wbase = wid * ROWS_PER_W
    iota = lax.iota(jnp.int32, 16)

    def stage(c, buf):
        b0 = wbase + c * CB
        pltpu.sync_copy(ids_hbm.at[pl.ds(b0, CB)], ids2_v.at[buf])
        pltpu.sync_copy(pos_hbm.at[pl.ds(b0, CB)], pidx2_v.at[buf])

        def tr_body(g, _):
            s = g * 16 + iota
            b = lax.div(s, SEQ)
            q = s - b * SEQ
            pvec = plsc.load_gather(pidx2_v.at[buf], [b, q])
            idxT2_v[buf, TRI, pl.ds(g * 16, 16)] = pvec
            for t in range(TRI):
                col = q * TRI + t
                vec = plsc.load_gather(ids2_v.at[buf], [b, col])
                idxT2_v[buf, t, pl.ds(g * 16, 16)] = vec
            return _

        lax.fori_loop(0, CS // 16, tr_body, None)
        pltpu.async_copy(post_hbm.at[idxT2_v.at[buf, TRI]], acc2_v.at[buf],
                         psem)

    stage(0, 0)

    def chunk_body(c, _):
        cur = lax.rem(c, 2)
        nxt = lax.rem(c + 1, 2)
        b0 = wbase + c * CB
        pltpu.make_async_copy(post_hbm.at[idxT2_v.at[cur, TRI]],
                              acc2_v.at[cur], psem).wait()
        descs = [
            pltpu.async_copy(tri_hbm.at[idxT2_v.at[cur, t]], acc2_v.at[cur],
                             addsem, add=True)
            for t in range(TRI)
        ]

        @pl.when(c > 0)
        def _():
            for r in range(CB):
                pltpu.make_async_copy(acc2_v.at[nxt, pl.ds(r * SEQ, SEQ)],
                                      out_hbm.at[b0 + r], outsem).wait()

        @pl.when(c < NCHUNK - 1)
        def _():
            stage(c + 1, nxt)

        for d in descs:
            d.wait()
        for r in range(CB):
            pltpu.async_copy(acc2_v.at[cur, pl.ds(r * SEQ, SEQ)],
                             out_hbm.at[b0 + r], outsem)
        return _

    lax.fori_loop(0, NCHUNK, chunk_body, None)
    last = lax.rem(NCHUNK - 1, 2)
    for r in range(CB):
        pltpu.make_async_copy(acc2_v.at[last, pl.ds(r * SEQ, SEQ)],
                              out_hbm.at[wbase + r], outsem).wait()


@jax.jit
def _run(input_ids, position_ids, tri_table, pos_table):
    mesh = plsc.VectorSubcoreMesh(core_axis_name="c", subcore_axis_name="s",
                                  num_cores=NC, num_subcores=NS)
    f = pl.kernel(
        _body,
        out_type=jax.ShapeDtypeStruct((B, SEQ, HIDDEN), jnp.float32),
        mesh=mesh,
        scratch_types=[
            pltpu.VMEM((2, CB, SEQ * TRI), jnp.int32),
            pltpu.VMEM((2, TRI + 1, CS), jnp.int32),
            pltpu.VMEM((2, CS, HIDDEN), jnp.float32),
            pltpu.VMEM((2, CB, SEQ), jnp.int32),
            pltpu.SemaphoreType.DMA,
            pltpu.SemaphoreType.DMA,
            pltpu.SemaphoreType.DMA,
        ],
        compiler_params=pltpu.CompilerParams(use_tc_tiling_on_sc=False,
                                             needs_layout_passes=False),
    )
    return f(input_ids, position_ids, tri_table, pos_table)


def kernel(input_ids, position_ids, token_type_ids, inputs_embeds,
           triletter_table, position_table):
    return _run(input_ids.astype(jnp.int32), position_ids.astype(jnp.int32),
                triletter_table, position_table)
